# baseline (speedup 1.0000x reference)
import functools

import jax
import jax.numpy as jnp
from jax import lax
from jax.experimental import pallas as pl
from jax.experimental.pallas import tpu as pltpu
from jax.experimental.pallas import tpu_sc as plsc

B = 16384
K = 64
N_VOCAB = 100000
NC = 2
NS = 16
NW = NC * NS
BPW = B // NW
L = 16
NPASS = K // NS
QTR = B // 4
CHUNK = 128
NCHUNK = BPW // CHUNK


def _gather_body(usrT_hbm, itmT_hbm, i_hbm, j_hbm, g_u, g_v,
                 frow, idxs, rowbuf0, rowbuf1, sem_w):
    core = lax.axis_index("c")
    s = lax.axis_index("s")
    zeros = jnp.zeros((L,), jnp.int32)
    UNROLL = 4

    def side(tbl, ids, gout):
        cp0 = pltpu.async_copy(tbl.at[pl.ds(s, 1), :], frow, sem_w)
        pltpu.sync_copy(ids, idxs)
        rowbufs = (rowbuf0, rowbuf1)
        wr = []
        for p in range(NPASS):
            f = p * NS + s
            if p == 0:
                cp0.wait()
            else:
                pltpu.sync_copy(tbl.at[pl.ds(f, 1), :], frow)
            for h in range(4):
                rowbuf = rowbufs[h % 2]
                if len(wr) >= 2:
                    wr.pop(0).wait()

                def grp(g4, _):
                    for u in range(UNROLL):
                        g = g4 * UNROLL + u
                        idxv = idxs[pl.ds(h * QTR + g * L, L)]
                        rowbuf[pl.ds(g * L, L)] = plsc.load_gather(
                            frow, [zeros, idxv])
                    return _
                lax.fori_loop(0, QTR // (L * UNROLL), grp, None)
                wr.append(pltpu.async_copy(
                    rowbuf, gout.at[pl.ds(f * B + h * QTR, QTR)], sem_w))
        for cp in wr:
            cp.wait()

    @pl.when(core == 0)
    def _():
        side(usrT_hbm, i_hbm, g_u)

    @pl.when(core == 1)
    def _():
        side(itmT_hbm, j_hbm, g_v)


def _dot_body(i_hbm, j_hbm, busr_hbm, bitm_hbm, g_u, g_v, c_hbm, out_hbm,
              idx_i, idx_j, ubuf, vbuf, bias_i, bias_j, c_v, out_v,
              sem_u, sem_v, sem_bi, sem_bj):
    core = lax.axis_index("c")
    s = lax.axis_index("s")
    wid = s * NC + core
    base = wid * BPW

    KH = K // 2
    lo_cp, hi_cp, bias_cp = [], [], []
    for f in range(KH):
        lo_cp.append(pltpu.async_copy(
            g_u.at[pl.ds(f * B + base, BPW)],
            ubuf.at[pl.ds(f * BPW, BPW)], sem_u))
        lo_cp.append(pltpu.async_copy(
            g_v.at[pl.ds(f * B + base, BPW)],
            vbuf.at[pl.ds(f * BPW, BPW)], sem_v))
    for f in range(KH, K):
        hi_cp.append(pltpu.async_copy(
            g_u.at[pl.ds(f * B + base, BPW)],
            ubuf.at[pl.ds(f * BPW, BPW)], sem_u))
        hi_cp.append(pltpu.async_copy(
            g_v.at[pl.ds(f * B + base, BPW)],
            vbuf.at[pl.ds(f * BPW, BPW)], sem_v))
    pltpu.sync_copy(i_hbm.at[pl.ds(base, BPW)], idx_i)
    pltpu.sync_copy(j_hbm.at[pl.ds(base, BPW)], idx_j)
    for q in range(NCHUNK):
        sl = q * CHUNK
        bias_cp.append(pltpu.async_copy(
            busr_hbm.at[idx_i.at[pl.ds(sl, CHUNK)]],
            bias_i.at[pl.ds(sl, CHUNK)], sem_bi))
        bias_cp.append(pltpu.async_copy(
            bitm_hbm.at[idx_j.at[pl.ds(sl, CHUNK)]],
            bias_j.at[pl.ds(sl, CHUNK)], sem_bj))
    pltpu.sync_copy(c_hbm, c_v)
    for cp in bias_cp:
        cp.wait()
    for cp in lo_cp:
        cp.wait()

    cvec = c_v[...]

    def grp_lo(g, _):
        acc = bias_i[pl.ds(g * L, L)] + bias_j[pl.ds(g * L, L)] + cvec
        for f in range(KH):
            acc += (ubuf[pl.ds(f * BPW + g * L, L)]
                    * vbuf[pl.ds(f * BPW + g * L, L)])
        out_v[pl.ds(g * L, L)] = acc
        return _

    lax.fori_loop(0, BPW // L, grp_lo, None)

    for cp in hi_cp:
        cp.wait()

    def grp_hi(g, _):
        acc = out_v[pl.ds(g * L, L)]
        for f in range(KH, K):
            acc += (ubuf[pl.ds(f * BPW + g * L, L)]
                    * vbuf[pl.ds(f * BPW + g * L, L)])
        out_v[pl.ds(g * L, L)] = acc
        return _

    lax.fori_loop(0, BPW // L, grp_hi, None)
    pltpu.sync_copy(out_v, out_hbm.at[pl.ds(base, BPW)])


@jax.jit
def kernel(i, j, y, busr, bitm, usr, itm, c):
    del y
    mesh = plsc.VectorSubcoreMesh(core_axis_name="c", subcore_axis_name="s")
    gather_call = pl.kernel(
        _gather_body,
        mesh=mesh,
        out_type=(jax.ShapeDtypeStruct((K * B,), jnp.float32),
                  jax.ShapeDtypeStruct((K * B,), jnp.float32)),
        compiler_params=pltpu.CompilerParams(
            needs_layout_passes=False, use_tc_tiling_on_sc=True),
        scratch_types=[
            pltpu.VMEM((1, N_VOCAB), jnp.float32),
            pltpu.VMEM((B,), jnp.int32),
            pltpu.VMEM((QTR,), jnp.float32),
            pltpu.VMEM((QTR,), jnp.float32),
            pltpu.SemaphoreType.DMA,
        ],
    )
    dot_call = pl.kernel(
        _dot_body,
        mesh=mesh,
        out_type=jax.ShapeDtypeStruct((B,), jnp.float32),
        compiler_params=pltpu.CompilerParams(
            needs_layout_passes=False, use_tc_tiling_on_sc=False),
        scratch_types=[
            pltpu.VMEM((BPW,), jnp.int32),
            pltpu.VMEM((BPW,), jnp.int32),
            pltpu.VMEM((K * BPW,), jnp.float32),
            pltpu.VMEM((K * BPW,), jnp.float32),
            pltpu.VMEM((BPW,), jnp.float32),
            pltpu.VMEM((BPW,), jnp.float32),
            pltpu.VMEM((L,), jnp.float32),
            pltpu.VMEM((BPW,), jnp.float32),
            pltpu.SemaphoreType.DMA,
            pltpu.SemaphoreType.DMA,
            pltpu.SemaphoreType.DMA,
            pltpu.SemaphoreType.DMA,
        ],
    )
    ii = i.astype(jnp.int32)
    jj = j.astype(jnp.int32)
    g_u, g_v = gather_call(usr.T, itm.T, ii, jj)
    c16 = jnp.broadcast_to(c, (L,))
    return dot_call(ii, jj, busr.reshape(-1), bitm.reshape(-1),
                    g_u, g_v, c16)


# speedup vs baseline: 1.0089x; 1.0089x over previous
import jax
import jax.numpy as jnp
from jax import lax
from jax.experimental import pallas as pl
from jax.experimental.pallas import tpu as pltpu
from jax.experimental.pallas import tpu_sc as plsc

B = 16384
K = 64
N_VOCAB = 100000
NC = 2
NS = 16
NW = NC * NS
BPW = B // NW
L = 16
NPASS = K // NS
QTR = B // 4
CHUNK = 128
NCHUNK = BPW // CHUNK


def _gather_body(usrT_hbm, itmT_hbm, i_hbm, j_hbm, g_u, g_v,
                 frow, idxs, rowbuf0, rowbuf1, sem_w):
    core = lax.axis_index("c")
    s = lax.axis_index("s")
    zeros = jnp.zeros((L,), jnp.int32)
    UNROLL = 4

    def side(tbl, ids, gout):
        cp0 = pltpu.async_copy(tbl.at[pl.ds(s, 1), :], frow, sem_w)
        pltpu.sync_copy(ids, idxs)
        rowbufs = (rowbuf0, rowbuf1)
        wr = []
        for p in range(NPASS):
            f = p * NS + s
            if p == 0:
                cp0.wait()
            else:
                pltpu.sync_copy(tbl.at[pl.ds(f, 1), :], frow)
            for h in range(4):
                rowbuf = rowbufs[h % 2]
                if len(wr) >= 2:
                    wr.pop(0).wait()

                def grp(g4, _):
                    for u in range(UNROLL):
                        g = g4 * UNROLL + u
                        idxv = idxs[pl.ds(h * QTR + g * L, L)]
                        rowbuf[pl.ds(g * L, L)] = plsc.load_gather(
                            frow, [zeros, idxv])
                    return _
                lax.fori_loop(0, QTR // (L * UNROLL), grp, None)
                wr.append(pltpu.async_copy(
                    rowbuf, gout.at[pl.ds(f * B + h * QTR, QTR)], sem_w))
        for cp in wr:
            cp.wait()

    @pl.when(core == 0)
    def _():
        side(usrT_hbm, i_hbm, g_u)

    @pl.when(core == 1)
    def _():
        side(itmT_hbm, j_hbm, g_v)


def _dot_body(i_hbm, j_hbm, busr_hbm, bitm_hbm, g_u, g_v, c_hbm, out_hbm,
              idx_i, idx_j, ubuf, vbuf, bias_i, bias_j, c_v, out_v,
              sem_u, sem_v, sem_bi, sem_bj):
    core = lax.axis_index("c")
    s = lax.axis_index("s")
    wid = s * NC + core
    base = wid * BPW

    copies = []
    for f in range(K):
        copies.append(pltpu.async_copy(
            g_u.at[pl.ds(f * B + base, BPW)],
            ubuf.at[pl.ds(f * BPW, BPW)], sem_u))
        copies.append(pltpu.async_copy(
            g_v.at[pl.ds(f * B + base, BPW)],
            vbuf.at[pl.ds(f * BPW, BPW)], sem_v))
    pltpu.sync_copy(i_hbm.at[pl.ds(base, BPW)], idx_i)
    pltpu.sync_copy(j_hbm.at[pl.ds(base, BPW)], idx_j)
    for q in range(NCHUNK):
        sl = q * CHUNK
        copies.append(pltpu.async_copy(
            busr_hbm.at[idx_i.at[pl.ds(sl, CHUNK)]],
            bias_i.at[pl.ds(sl, CHUNK)], sem_bi))
        copies.append(pltpu.async_copy(
            bitm_hbm.at[idx_j.at[pl.ds(sl, CHUNK)]],
            bias_j.at[pl.ds(sl, CHUNK)], sem_bj))
    pltpu.sync_copy(c_hbm, c_v)
    for cp in copies:
        cp.wait()

    cvec = c_v[...]

    def grp(g, _):
        acc = bias_i[pl.ds(g * L, L)] + bias_j[pl.ds(g * L, L)] + cvec
        for f in range(K):
            acc += (ubuf[pl.ds(f * BPW + g * L, L)]
                    * vbuf[pl.ds(f * BPW + g * L, L)])
        out_v[pl.ds(g * L, L)] = acc
        return _

    lax.fori_loop(0, BPW // L, grp, None)
    pltpu.sync_copy(out_v, out_hbm.at[pl.ds(base, BPW)])


@jax.jit
def kernel(i, j, y, busr, bitm, usr, itm, c):
    del y
    mesh = plsc.VectorSubcoreMesh(core_axis_name="c", subcore_axis_name="s")
    gather_call = pl.kernel(
        _gather_body,
        mesh=mesh,
        out_type=(jax.ShapeDtypeStruct((K * B,), jnp.float32),
                  jax.ShapeDtypeStruct((K * B,), jnp.float32)),
        compiler_params=pltpu.CompilerParams(
            needs_layout_passes=False, use_tc_tiling_on_sc=True),
        scratch_types=[
            pltpu.VMEM((1, N_VOCAB), jnp.float32),
            pltpu.VMEM((B,), jnp.int32),
            pltpu.VMEM((QTR,), jnp.float32),
            pltpu.VMEM((QTR,), jnp.float32),
            pltpu.SemaphoreType.DMA,
        ],
    )
    dot_call = pl.kernel(
        _dot_body,
        mesh=mesh,
        out_type=jax.ShapeDtypeStruct((B,), jnp.float32),
        compiler_params=pltpu.CompilerParams(
            needs_layout_passes=False, use_tc_tiling_on_sc=False),
        scratch_types=[
            pltpu.VMEM((BPW,), jnp.int32),
            pltpu.VMEM((BPW,), jnp.int32),
            pltpu.VMEM((K * BPW,), jnp.float32),
            pltpu.VMEM((K * BPW,), jnp.float32),
            pltpu.VMEM((BPW,), jnp.float32),
            pltpu.VMEM((BPW,), jnp.float32),
            pltpu.VMEM((L,), jnp.float32),
            pltpu.VMEM((BPW,), jnp.float32),
            pltpu.SemaphoreType.DMA,
            pltpu.SemaphoreType.DMA,
            pltpu.SemaphoreType.DMA,
            pltpu.SemaphoreType.DMA,
        ],
    )
    ii = i.astype(jnp.int32)
    jj = j.astype(jnp.int32)
    g_u, g_v = gather_call(usr.T, itm.T, ii, jj)
    c16 = jnp.broadcast_to(c, (L,))
    return dot_call(ii, jj, busr.reshape(-1), bitm.reshape(-1),
                    g_u, g_v, c16)
